# baseline (speedup 1.0000x reference)
import jax
import jax.numpy as jnp
from jax import lax
from jax.experimental import pallas as pl
from jax.experimental.pallas import tpu as pltpu
from jax.experimental.pallas import tpu_sc as plsc

N_NODES = 10000
NP = 10240
TPN = 320
E = 320000
E_SORT = 321536
E_PHYS = E_SORT + 2048
C = 224
CH = C // 2
HID = 64
PADV = NP - 1


def _in_body(x_ref, w_ref, b_ref, h_ref):
    h_ref[...] = jax.nn.relu(
        jnp.dot(x_ref[...], w_ref[...], preferred_element_type=jnp.float32)
        + b_ref[...])


def _proj_body(h_ref, ws_ref, wd_ref, b_ref, hs_ref, hd_ref):
    h = h_ref[...]
    hs_ref[...] = jnp.dot(h, ws_ref[...], preferred_element_type=jnp.float32)
    hd_ref[...] = jnp.dot(h, wd_ref[...], preferred_element_type=jnp.float32) + b_ref[...]


def _eproj_body(ea_ref, w_ref, o0, o1, o2, o3, o4):
    ea = ea_ref[...]
    outs = (o0, o1, o2, o3, o4)
    for l in range(5):
        outs[l][...] = jnp.dot(ea, w_ref[l], preferred_element_type=jnp.float32)


def _post_body(h_ref, s_ref, m_ref, deg_ref, ph_ref, pm_ref, px_ref, b_ref, o_ref):
    h = h_ref[...]
    inv = 1.0 / jnp.maximum(deg_ref[...], 1.0)
    mean = s_ref[...] * inv
    o = (jnp.dot(h, ph_ref[...], preferred_element_type=jnp.float32)
         + jnp.dot(mean, pm_ref[...], preferred_element_type=jnp.float32)
         + jnp.dot(m_ref[...], px_ref[...], preferred_element_type=jnp.float32)
         + b_ref[...])
    o_ref[...] = o + h


def _readout_body(h_ref, w1_ref, b1_ref, w2_ref, b2_ref, o_ref):
    rows = lax.broadcasted_iota(jnp.int32, (NP, 1), 0)
    valid = rows < N_NODES
    h = h_ref[...]
    hs = jnp.where(valid, h, 0.0)
    hm = jnp.where(valid, h, -jnp.inf)
    s = jnp.sum(hs, axis=0, keepdims=True)
    mx = jnp.max(hm, axis=0, keepdims=True)
    r = jnp.concatenate([s, s * (1.0 / N_NODES), mx], axis=1)
    o = jax.nn.relu(
        jnp.dot(r, w1_ref[...], preferred_element_type=jnp.float32) + b1_ref[...])
    o_ref[...] = jnp.dot(o, w2_ref[...], preferred_element_type=jnp.float32) + b2_ref[...]


def _tc_in(x_pad, w, b):
    return pl.pallas_call(
        _in_body,
        out_shape=jax.ShapeDtypeStruct((NP, HID), jnp.float32),
    )(x_pad, w, b[None, :])


def _tc_proj(h, ws, wd, b):
    return pl.pallas_call(
        _proj_body,
        out_shape=[jax.ShapeDtypeStruct((NP, HID), jnp.float32)] * 2,
    )(h, ws, wd, b[None, :])


def _tc_eproj(ea_phys, w_stack):
    blk = 2048
    grid = E_PHYS // blk
    return pl.pallas_call(
        _eproj_body,
        grid=(grid,),
        in_specs=[
            pl.BlockSpec((blk, 16), lambda i: (i, 0)),
            pl.BlockSpec((5, 16, HID), lambda i: (0, 0, 0)),
        ],
        out_specs=[pl.BlockSpec((blk, HID), lambda i: (i, 0))] * 5,
        out_shape=[jax.ShapeDtypeStruct((E_PHYS, HID), jnp.float32)] * 5,
    )(ea_phys, w_stack)


def _tc_post(h, s, m, deg, ph, pm, px, b):
    return pl.pallas_call(
        _post_body,
        out_shape=jax.ShapeDtypeStruct((NP, HID), jnp.float32),
    )(h, s, m, deg, ph, pm, px, b[None, :])


def _tc_readout(h, w1, b1, w2, b2):
    return pl.pallas_call(
        _readout_body,
        out_shape=jax.ShapeDtypeStruct((1, w2.shape[1]), jnp.float32),
    )(h, w1, b1[None, :], w2, b2[None, :])


_MESH = plsc.VectorSubcoreMesh(core_axis_name="c", subcore_axis_name="s")


def _make_sc_edge(want_deg):
    out_type = [jax.ShapeDtypeStruct((NP, HID), jnp.float32)] * 2
    if want_deg:
        out_type = out_type + [jax.ShapeDtypeStruct((NP,), jnp.float32)]
    scratch = (
        [pltpu.VMEM((48,), jnp.int32)]
        + [pltpu.VMEM((C,), jnp.int32)] * 4
        + [pltpu.VMEM((C, HID), jnp.float32)] * 4
        + [pltpu.VMEM((TPN + 1, HID), jnp.float32)] * 3
        + [pltpu.VMEM((336,), jnp.float32)]
        + [pltpu.SemaphoreType.DMA] * 10
    )

    def body(dst_hbm, src_hbm, hs_hbm, hd_hbm, ep_hbm, bounds_hbm,
             osum_hbm, omax_hbm, *rest):
        if want_deg:
            odeg_hbm = rest[0]
            rest = rest[1:]
        (bv, didx0, didx1, sidx0, sidx1, hsv0, hsv1, epv0, epv1,
         hdv, accs, accm, degv,
         sd0, sd1, ss0, ss1, ga0, gb0, ge0, ga1, gb1, ge1) = rest
        didx = (didx0, didx1)
        sidx = (sidx0, sidx1)
        hsv = (hsv0, hsv1)
        epv = (epv0, epv1)
        dsem = (sd0, sd1)
        ssem = (ss0, ss1)
        gsem = ((ga0, gb0, ge0), (ga1, gb1, ge1))

        wid = lax.axis_index("s") * 2 + lax.axis_index("c")
        n0 = wid * TPN
        pltpu.sync_copy(bounds_hbm, bv)
        bwin = bv[pl.ds(wid, 16)]
        e0 = bwin[0]
        e1 = bwin[1]
        e0a = (e0 // 8) * 8
        nch = (e1 - e0a + (C - 1)) // C

        zero16 = jnp.zeros((16,), jnp.float32)

        @pl.loop(0, TPN + 1)
        def _(r):
            for j in range(4):
                accs[r, pl.ds(j * 16, 16)] = zero16
                accm[r, pl.ds(j * 16, 16)] = zero16

        if want_deg:
            @pl.loop(0, 336 // 16)
            def _(r):
                degv[pl.ds(r * 16, 16)] = zero16

        pltpu.sync_copy(hd_hbm.at[pl.ds(n0, TPN)], hdv.at[pl.ds(0, TPN)])
        for j in range(4):
            hdv[TPN, pl.ds(j * 16, 16)] = zero16

        def issue_idx(k, b):
            ec = e0a + k * C
            pltpu.async_copy(dst_hbm.at[pl.ds(ec, C)], didx[b], dsem[b])
            pltpu.async_copy(src_hbm.at[pl.ds(ec, C)], sidx[b], ssem[b])

        def wait_idx(b):
            pltpu.make_async_copy(dst_hbm.at[pl.ds(0, C)], didx[b], dsem[b]).wait()
            pltpu.make_async_copy(src_hbm.at[pl.ds(0, C)], sidx[b], ssem[b]).wait()

        def issue_gather(k, b):
            ec = e0a + k * C
            pltpu.async_copy(hs_hbm.at[sidx[b].at[pl.ds(0, CH)]],
                             hsv[b].at[pl.ds(0, CH)], gsem[b][0])
            pltpu.async_copy(hs_hbm.at[sidx[b].at[pl.ds(CH, CH)]],
                             hsv[b].at[pl.ds(CH, CH)], gsem[b][1])
            pltpu.async_copy(ep_hbm.at[pl.ds(ec, C)], epv[b], gsem[b][2])

        def wait_gather(b):
            pltpu.make_async_copy(hs_hbm.at[sidx[b].at[pl.ds(0, CH)]],
                                  hsv[b].at[pl.ds(0, CH)], gsem[b][0]).wait()
            pltpu.make_async_copy(hs_hbm.at[sidx[b].at[pl.ds(CH, CH)]],
                                  hsv[b].at[pl.ds(CH, CH)], gsem[b][1]).wait()
            pltpu.make_async_copy(ep_hbm.at[pl.ds(0, C)], epv[b], gsem[b][2]).wait()

        def compute(b):
            @pl.loop(0, C // 16)
            def _(i2):
                dvec = didx[b][pl.ds(i2 * 16, 16)]
                dlv = dvec - n0
                okv = (dlv >= 0) & (dlv < TPN)
                rv = jnp.where(okv, dlv, TPN)
                if want_deg:
                    ones = jnp.where(okv, 1.0, 0.0).astype(jnp.float32)
                    plsc.addupdate_scatter(degv, [rv], ones)
                for lane in range(16):
                    r = rv[lane]
                    ei = i2 * 16 + lane
                    for j in range(4):
                        sl = pl.ds(j * 16, 16)
                        v = hsv[b][ei, sl] + epv[b][ei, sl] + hdv[r, sl]
                        v = jnp.maximum(v, 0.0)
                        accs[r, sl] += v
                        accm[r, sl] = jnp.maximum(accm[r, sl], v)

        def process(k, cur, nxt):
            @pl.when(k + 1 < nch)
            def _():
                issue_idx(k + 1, nxt)
            wait_gather(cur)

            @pl.when(k + 1 < nch)
            def _():
                wait_idx(nxt)
                issue_gather(k + 1, nxt)
            compute(cur)

        @pl.when(nch > 0)
        def _():
            issue_idx(0, 0)
            wait_idx(0)
            issue_gather(0, 0)

            def pair(p, carry):
                k0 = 2 * p
                process(k0, 0, 1)

                @pl.when(k0 + 1 < nch)
                def _():
                    process(k0 + 1, 1, 0)
                return carry

            lax.fori_loop(0, (nch + 1) // 2, pair, 0)

        pltpu.sync_copy(accs.at[pl.ds(0, TPN)], osum_hbm.at[pl.ds(n0, TPN)])
        pltpu.sync_copy(accm.at[pl.ds(0, TPN)], omax_hbm.at[pl.ds(n0, TPN)])
        if want_deg:
            pltpu.sync_copy(degv.at[pl.ds(0, TPN)], odeg_hbm.at[pl.ds(n0, TPN)])

    cp = pltpu.CompilerParams(needs_layout_passes=False,
                              use_tc_tiling_on_sc=False)
    return pl.kernel(body, out_type=out_type, mesh=_MESH, scratch_types=scratch,
                     compiler_params=cp)


_sc_edge_deg = _make_sc_edge(True)
_sc_edge = _make_sc_edge(False)


def kernel(x, edge_index, edge_attr, params):
    src = edge_index[0].astype(jnp.int32)
    dst = edge_index[1].astype(jnp.int32)

    npad = E_SORT - E
    dst_pad = jnp.concatenate([dst, jnp.full((npad,), PADV, jnp.int32)])
    eidx = jnp.arange(E_SORT, dtype=jnp.int32)
    dst_s, perm = lax.sort((dst_pad, eidx), num_keys=1)
    src_pad = jnp.concatenate([src, jnp.zeros((npad,), jnp.int32)])
    src_s = jnp.take(src_pad, perm)
    ea_pad = jnp.concatenate(
        [edge_attr, jnp.zeros((npad, edge_attr.shape[1]), jnp.float32)])
    ea_s = jnp.take(ea_pad, perm, axis=0)
    slack = E_PHYS - E_SORT
    dst_phys = jnp.concatenate([dst_s, jnp.full((slack,), PADV, jnp.int32)])
    src_phys = jnp.concatenate([src_s, jnp.zeros((slack,), jnp.int32)])
    ea_phys = jnp.concatenate(
        [ea_s, jnp.zeros((slack, edge_attr.shape[1]), jnp.float32)])

    bounds = jnp.searchsorted(
        dst_s, jnp.arange(33, dtype=jnp.int32) * TPN).astype(jnp.int32)
    bounds = jnp.concatenate([bounds, jnp.zeros((15,), jnp.int32)])

    x_pad = jnp.concatenate(
        [x, jnp.zeros((NP - N_NODES, x.shape[1]), jnp.float32)])

    layers = params["layers"]
    we_stack = jnp.stack([l["pre_W"][128:144] for l in layers])
    eprojs = _tc_eproj(ea_phys, we_stack)

    h = _tc_in(x_pad, params["in_W"], params["in_b"])
    deg = None
    for l in range(5):
        lay = layers[l]
        hs, hd = _tc_proj(h, lay["pre_W"][:64], lay["pre_W"][64:128], lay["pre_b"])
        if l == 0:
            ssum, smax, deg = _sc_edge_deg(
                dst_phys, src_phys, hs, hd, eprojs[l], bounds)
            deg = deg[:, None]
        else:
            ssum, smax = _sc_edge(
                dst_phys, src_phys, hs, hd, eprojs[l], bounds)
        pw = lay["post_W"]
        h = _tc_post(h, ssum, smax, deg,
                     pw[:64], pw[64:128], pw[128:192], lay["post_b"])

    return _tc_readout(h, params["ro1_W"], params["ro1_b"],
                       params["ro2_W"], params["ro2_b"])


# speedup vs baseline: 1.0047x; 1.0047x over previous
import jax
import jax.numpy as jnp
from jax import lax
from jax.experimental import pallas as pl
from jax.experimental.pallas import tpu as pltpu
from jax.experimental.pallas import tpu_sc as plsc

N_NODES = 10000
NP = 10240
TPN = 320
E = 320000
E_SORT = 321536
E_PHYS = E_SORT + 2048
C = 240
HID = 64
PADV = NP - 1


def _in_body(x_ref, w_ref, b_ref, h_ref):
    h_ref[...] = jax.nn.relu(
        jnp.dot(x_ref[...], w_ref[...], preferred_element_type=jnp.float32)
        + b_ref[...])


def _inproj_body(x_ref, w_ref, b_ref, ws_ref, wd_ref, pb_ref,
                 h_ref, hs_ref, hd_ref):
    h = jax.nn.relu(
        jnp.dot(x_ref[...], w_ref[...], preferred_element_type=jnp.float32)
        + b_ref[...])
    h_ref[...] = h
    hs_ref[...] = jnp.dot(h, ws_ref[...], preferred_element_type=jnp.float32)
    hd_ref[...] = jnp.dot(h, wd_ref[...], preferred_element_type=jnp.float32) + pb_ref[...]


def _postproj_body(h_ref, s_ref, m_ref, deg_ref, ph_ref, pm_ref, px_ref, b_ref,
                   ws_ref, wd_ref, pb_ref, o_ref, hs_ref, hd_ref):
    h = h_ref[...]
    inv = 1.0 / jnp.maximum(deg_ref[...], 1.0)
    mean = s_ref[...] * inv
    o = (jnp.dot(h, ph_ref[...], preferred_element_type=jnp.float32)
         + jnp.dot(mean, pm_ref[...], preferred_element_type=jnp.float32)
         + jnp.dot(m_ref[...], px_ref[...], preferred_element_type=jnp.float32)
         + b_ref[...])
    o = o + h
    o_ref[...] = o
    hs_ref[...] = jnp.dot(o, ws_ref[...], preferred_element_type=jnp.float32)
    hd_ref[...] = jnp.dot(o, wd_ref[...], preferred_element_type=jnp.float32) + pb_ref[...]


def _eproj_body(ea_ref, w_ref, o0, o1, o2, o3, o4):
    ea = ea_ref[...]
    outs = (o0, o1, o2, o3, o4)
    for l in range(5):
        outs[l][...] = jnp.dot(ea, w_ref[l], preferred_element_type=jnp.float32)


def _post_body(h_ref, s_ref, m_ref, deg_ref, ph_ref, pm_ref, px_ref, b_ref, o_ref):
    h = h_ref[...]
    inv = 1.0 / jnp.maximum(deg_ref[...], 1.0)
    mean = s_ref[...] * inv
    o = (jnp.dot(h, ph_ref[...], preferred_element_type=jnp.float32)
         + jnp.dot(mean, pm_ref[...], preferred_element_type=jnp.float32)
         + jnp.dot(m_ref[...], px_ref[...], preferred_element_type=jnp.float32)
         + b_ref[...])
    o_ref[...] = o + h


def _readout_body(h_ref, w1_ref, b1_ref, w2_ref, b2_ref, o_ref):
    rows = lax.broadcasted_iota(jnp.int32, (NP, 1), 0)
    valid = rows < N_NODES
    h = h_ref[...]
    hs = jnp.where(valid, h, 0.0)
    hm = jnp.where(valid, h, -jnp.inf)
    s = jnp.sum(hs, axis=0, keepdims=True)
    mx = jnp.max(hm, axis=0, keepdims=True)
    r = jnp.concatenate([s, s * (1.0 / N_NODES), mx], axis=1)
    o = jax.nn.relu(
        jnp.dot(r, w1_ref[...], preferred_element_type=jnp.float32) + b1_ref[...])
    o_ref[...] = jnp.dot(o, w2_ref[...], preferred_element_type=jnp.float32) + b2_ref[...]


def _tc_inproj(x_pad, w, b, ws, wd, pb):
    return pl.pallas_call(
        _inproj_body,
        out_shape=[jax.ShapeDtypeStruct((NP, HID), jnp.float32)] * 3,
    )(x_pad, w, b[None, :], ws, wd, pb[None, :])


def _tc_postproj(h, s, m, deg, ph, pm, px, b, ws, wd, pb):
    return pl.pallas_call(
        _postproj_body,
        out_shape=[jax.ShapeDtypeStruct((NP, HID), jnp.float32)] * 3,
    )(h, s, m, deg, ph, pm, px, b[None, :], ws, wd, pb[None, :])


def _tc_eproj(ea_phys, w_stack):
    blk = 2048
    grid = E_PHYS // blk
    return pl.pallas_call(
        _eproj_body,
        grid=(grid,),
        in_specs=[
            pl.BlockSpec((blk, 16), lambda i: (i, 0)),
            pl.BlockSpec((5, 16, HID), lambda i: (0, 0, 0)),
        ],
        out_specs=[pl.BlockSpec((blk, HID), lambda i: (i, 0))] * 5,
        out_shape=[jax.ShapeDtypeStruct((E_PHYS, HID), jnp.float32)] * 5,
    )(ea_phys, w_stack)


def _tc_post(h, s, m, deg, ph, pm, px, b):
    return pl.pallas_call(
        _post_body,
        out_shape=jax.ShapeDtypeStruct((NP, HID), jnp.float32),
    )(h, s, m, deg, ph, pm, px, b[None, :])


def _tc_readout(h, w1, b1, w2, b2):
    return pl.pallas_call(
        _readout_body,
        out_shape=jax.ShapeDtypeStruct((1, w2.shape[1]), jnp.float32),
    )(h, w1, b1[None, :], w2, b2[None, :])


_MESH = plsc.VectorSubcoreMesh(core_axis_name="c", subcore_axis_name="s")


def _make_sc_edge(want_deg):
    out_type = [jax.ShapeDtypeStruct((NP, HID), jnp.float32)] * 2
    if want_deg:
        out_type = out_type + [jax.ShapeDtypeStruct((NP,), jnp.float32)]
    scratch = (
        [pltpu.VMEM((48,), jnp.int32)]
        + [pltpu.VMEM((2, C), jnp.int32)] * 2
        + [pltpu.VMEM((C, HID), jnp.float32)] * 4
        + [pltpu.VMEM((TPN + 1, HID), jnp.float32)] * 3
        + [pltpu.VMEM((336,), jnp.float32)]
        + [pltpu.SemaphoreType.DMA] * 6
    )

    def body(ds_hbm, hs_hbm, hd_hbm, ep_hbm, bounds_hbm,
             osum_hbm, omax_hbm, *rest):
        if want_deg:
            odeg_hbm = rest[0]
            rest = rest[1:]
        (bv, idx0, idx1, hsv0, hsv1, epv0, epv1,
         hdv, accs, accm, degv,
         si0, si1, gh0, gh1, ge0, ge1) = rest
        idx = (idx0, idx1)
        hsv = (hsv0, hsv1)
        epv = (epv0, epv1)
        isem = (si0, si1)
        hsem = (gh0, gh1)
        esem = (ge0, ge1)

        wid = lax.axis_index("s") * 2 + lax.axis_index("c")
        n0 = wid * TPN
        pltpu.sync_copy(bounds_hbm, bv)
        bwin = bv[pl.ds(wid, 16)]
        e0 = bwin[0]
        e1 = bwin[1]
        e0a = (e0 // 8) * 8
        nch = (e1 - e0a + (C - 1)) // C

        zero16 = jnp.zeros((16,), jnp.float32)

        @pl.loop(0, TPN + 1)
        def _(r):
            for j in range(4):
                accs[r, pl.ds(j * 16, 16)] = zero16
                accm[r, pl.ds(j * 16, 16)] = zero16

        if want_deg:
            @pl.loop(0, 336 // 16)
            def _(r):
                degv[pl.ds(r * 16, 16)] = zero16

        pltpu.sync_copy(hd_hbm.at[pl.ds(n0, TPN)], hdv.at[pl.ds(0, TPN)])
        for j in range(4):
            hdv[TPN, pl.ds(j * 16, 16)] = zero16

        def issue_idx(k, b):
            ec = e0a + k * C
            pltpu.async_copy(ds_hbm.at[:, pl.ds(ec, C)], idx[b], isem[b])

        def wait_idx(b):
            pltpu.make_async_copy(ds_hbm.at[:, pl.ds(0, C)], idx[b], isem[b]).wait()

        def issue_gather(k, b):
            ec = e0a + k * C
            pltpu.async_copy(hs_hbm.at[idx[b].at[1]], hsv[b], hsem[b])
            pltpu.async_copy(ep_hbm.at[pl.ds(ec, C)], epv[b], esem[b])

        def wait_gather(b):
            pltpu.make_async_copy(hs_hbm.at[idx[b].at[1]], hsv[b], hsem[b]).wait()
            pltpu.make_async_copy(ep_hbm.at[pl.ds(0, C)], epv[b], esem[b]).wait()

        def compute(b):
            @pl.loop(0, C // 16)
            def _(i2):
                dvec = idx[b][0, pl.ds(i2 * 16, 16)]
                dlv = dvec - n0
                okv = (dlv >= 0) & (dlv < TPN)
                rv = jnp.where(okv, dlv, TPN)
                if want_deg:
                    ones = jnp.where(okv, 1.0, 0.0).astype(jnp.float32)
                    plsc.addupdate_scatter(degv, [rv], ones)
                for lane in range(16):
                    r = rv[lane]
                    ei = i2 * 16 + lane
                    for j in range(4):
                        sl = pl.ds(j * 16, 16)
                        v = hsv[b][ei, sl] + epv[b][ei, sl] + hdv[r, sl]
                        v = jnp.maximum(v, 0.0)
                        accs[r, sl] += v
                        accm[r, sl] = jnp.maximum(accm[r, sl], v)

        def process(k, cur, nxt):
            @pl.when(k + 1 < nch)
            def _():
                issue_idx(k + 1, nxt)
            wait_gather(cur)

            @pl.when(k + 1 < nch)
            def _():
                wait_idx(nxt)
                issue_gather(k + 1, nxt)
            compute(cur)

        @pl.when(nch > 0)
        def _():
            issue_idx(0, 0)
            wait_idx(0)
            issue_gather(0, 0)

            def pair(p, carry):
                k0 = 2 * p
                process(k0, 0, 1)

                @pl.when(k0 + 1 < nch)
                def _():
                    process(k0 + 1, 1, 0)
                return carry

            lax.fori_loop(0, (nch + 1) // 2, pair, 0)

        pltpu.sync_copy(accs.at[pl.ds(0, TPN)], osum_hbm.at[pl.ds(n0, TPN)])
        pltpu.sync_copy(accm.at[pl.ds(0, TPN)], omax_hbm.at[pl.ds(n0, TPN)])
        if want_deg:
            pltpu.sync_copy(degv.at[pl.ds(0, TPN)], odeg_hbm.at[pl.ds(n0, TPN)])

    cp = pltpu.CompilerParams(needs_layout_passes=False,
                              use_tc_tiling_on_sc=False)
    return pl.kernel(body, out_type=out_type, mesh=_MESH, scratch_types=scratch,
                     compiler_params=cp)


_sc_edge_deg = _make_sc_edge(True)
_sc_edge = _make_sc_edge(False)


def kernel(x, edge_index, edge_attr, params):
    src = edge_index[0].astype(jnp.int32)
    dst = edge_index[1].astype(jnp.int32)

    npad = E_SORT - E
    dst_pad = jnp.concatenate([dst, jnp.full((npad,), PADV, jnp.int32)])
    eidx = jnp.arange(E_SORT, dtype=jnp.int32)
    dst_s, perm = lax.sort((dst_pad, eidx), num_keys=1)
    src_pad = jnp.concatenate([src, jnp.zeros((npad,), jnp.int32)])
    src_s = jnp.take(src_pad, perm)
    ea_pad = jnp.concatenate(
        [edge_attr, jnp.zeros((npad, edge_attr.shape[1]), jnp.float32)])
    ea_s = jnp.take(ea_pad, perm, axis=0)
    slack = E_PHYS - E_SORT
    dst_phys = jnp.concatenate([dst_s, jnp.full((slack,), PADV, jnp.int32)])
    src_phys = jnp.concatenate([src_s, jnp.zeros((slack,), jnp.int32)])
    ea_phys = jnp.concatenate(
        [ea_s, jnp.zeros((slack, edge_attr.shape[1]), jnp.float32)])

    bounds = jnp.searchsorted(
        dst_s, jnp.arange(33, dtype=jnp.int32) * TPN).astype(jnp.int32)
    bounds = jnp.concatenate([bounds, jnp.zeros((15,), jnp.int32)])
    dstsrc = jnp.stack([dst_phys, src_phys])

    x_pad = jnp.concatenate(
        [x, jnp.zeros((NP - N_NODES, x.shape[1]), jnp.float32)])

    layers = params["layers"]
    we_stack = jnp.stack([l["pre_W"][128:144] for l in layers])
    eprojs = _tc_eproj(ea_phys, we_stack)

    h, hs, hd = _tc_inproj(x_pad, params["in_W"], params["in_b"],
                           layers[0]["pre_W"][:64], layers[0]["pre_W"][64:128],
                           layers[0]["pre_b"])
    deg = None
    for l in range(5):
        lay = layers[l]
        if l == 0:
            ssum, smax, deg = _sc_edge_deg(
                dstsrc, hs, hd, eprojs[l], bounds)
            deg = deg[:, None]
        else:
            ssum, smax = _sc_edge(
                dstsrc, hs, hd, eprojs[l], bounds)
        pw = lay["post_W"]
        if l < 4:
            nxt = layers[l + 1]
            h, hs, hd = _tc_postproj(h, ssum, smax, deg,
                                     pw[:64], pw[64:128], pw[128:192],
                                     lay["post_b"], nxt["pre_W"][:64],
                                     nxt["pre_W"][64:128], nxt["pre_b"])
        else:
            h = _tc_post(h, ssum, smax, deg,
                         pw[:64], pw[64:128], pw[128:192], lay["post_b"])

    return _tc_readout(h, params["ro1_W"], params["ro1_b"],
                       params["ro2_W"], params["ro2_b"])
